# baseline (speedup 1.0000x reference)
import jax
import jax.numpy as jnp
from jax.experimental import pallas as pl
from jax.experimental.pallas import tpu as pltpu

NSQ = 8
TOPK = 8
BN_EPS = 1e-5


def _body(x_ref, gw_ref, uw_ref, dw_ref, rx_ref, ry_ref, ue_ref, de_ref,
          o_ref):
    f = pl.program_id(1)
    x = x_ref[...]

    @pl.when(f == 0)
    def _router_and_experts():
      bn_scale = 1.0 / jnp.sqrt(1.0 + BN_EPS)
      tm_full = x.shape[0]
      rc = min(256, tm_full)
      for kc in range(tm_full // rc):
        xs = x[kc * rc:(kc + 1) * rc, :]
        lx = jax.lax.dot_general(
            xs, rx_ref[...], (((1,), (1,)), ((), ())),
            preferred_element_type=jnp.float32) * bn_scale
        ly = jax.lax.dot_general(
            xs, ry_ref[...], (((1,), (1,)), ((), ())),
            preferred_element_type=jnp.float32) * bn_scale
        mx = jnp.max(lx, axis=-1, keepdims=True)
        lpx = (lx - mx) - jnp.log(
            jnp.sum(jnp.exp(lx - mx), axis=-1, keepdims=True))
        my = jnp.max(ly, axis=-1, keepdims=True)
        lpy = (ly - my) - jnp.log(
            jnp.sum(jnp.exp(ly - my), axis=-1, keepdims=True))
        c = jnp.concatenate(
            [lpx[:, i:i + 1] + lpy for i in range(NSQ)], axis=-1)
        eidx = jax.lax.broadcasted_iota(jnp.int32, (rc, NSQ * NSQ), 1)
        sel = jnp.zeros((rc, NSQ * NSQ), dtype=jnp.bool_)
        work = c
        for _ in range(TOPK):
            m = jnp.max(work, axis=-1, keepdims=True)
            eq = work == m
            minidx = jnp.min(jnp.where(eq, eidx, NSQ * NSQ),
                             axis=-1, keepdims=True)
            first = eidx == minidx
            sel = sel | first
            work = jnp.where(first, -jnp.inf, work)
        rw = jnp.where(sel, jnp.exp(c), 0.0)
        ax = jax.lax.dot_general(
            xs, ue_ref[...], (((1,), (1,)), ((), ())),
            preferred_element_type=jnp.float32)
        w = jax.nn.silu(ax) * rw
        o_ref[kc * rc:(kc + 1) * rc, :] = jnp.dot(
            w.astype(jnp.bfloat16), de_ref[...],
            preferred_element_type=jnp.float32)

    g = jax.lax.dot_general(x, gw_ref[...], (((1,), (1,)), ((), ())),
                            preferred_element_type=jnp.float32)
    u = jax.lax.dot_general(x, uw_ref[...], (((1,), (1,)), ((), ())),
                            preferred_element_type=jnp.float32)
    h = jax.nn.silu(g) * u
    o_ref[...] += jax.lax.dot_general(h.astype(jnp.bfloat16), dw_ref[...],
                                      (((1,), (1,)), ((), ())),
                                      preferred_element_type=jnp.float32)


def kernel(hidden_states, gate_proj_w, up_proj_w, down_proj_w,
           router_gate_x_w, router_gate_y_w, up_embed, down_embed):
    bsz, seq, d = hidden_states.shape
    t = bsz * seq
    ff = gate_proj_w.shape[0]
    x = hidden_states.reshape(t, d).astype(jnp.bfloat16)
    gw = gate_proj_w.astype(jnp.bfloat16)
    uw = up_proj_w.astype(jnp.bfloat16)
    dw = down_proj_w.astype(jnp.bfloat16)
    rx = router_gate_x_w.astype(jnp.bfloat16)
    ry = router_gate_y_w.astype(jnp.bfloat16)
    ue = up_embed.astype(jnp.bfloat16)
    de = down_embed.astype(jnp.bfloat16)

    tm = min(2048, t)
    fk = min(128, ff)
    n_t = t // tm
    n_ff = ff // fk

    out = pl.pallas_call(
        _body,
        grid=(n_t, n_ff),
        in_specs=[
            pl.BlockSpec((tm, d), lambda i, j: (i, 0)),
            pl.BlockSpec((fk, d), lambda i, j: (j, 0)),
            pl.BlockSpec((fk, d), lambda i, j: (j, 0)),
            pl.BlockSpec((d, fk), lambda i, j: (0, j)),
            pl.BlockSpec((NSQ, d), lambda i, j: (0, 0)),
            pl.BlockSpec((NSQ, d), lambda i, j: (0, 0)),
            pl.BlockSpec((NSQ * NSQ, d), lambda i, j: (0, 0)),
            pl.BlockSpec((NSQ * NSQ, d), lambda i, j: (0, 0)),
        ],
        out_specs=pl.BlockSpec((tm, d), lambda i, j: (i, 0)),
        out_shape=jax.ShapeDtypeStruct((t, d), jnp.float32),
        compiler_params=pltpu.CompilerParams(
            dimension_semantics=("parallel", "arbitrary"),
            vmem_limit_bytes=64 * 1024 * 1024,
        ),
    )(x, gw, uw, dw, rx, ry, ue, de)
    return out.reshape(bsz, seq, d)


# speedup vs baseline: 1.7131x; 1.7131x over previous
import jax
import jax.numpy as jnp
from jax.experimental import pallas as pl
from jax.experimental.pallas import tpu as pltpu

NSQ = 8
TOPK = 8
BN_EPS = 1e-5


def _body(x_ref, gw_ref, uw_ref, dw_ref, rx_ref, ry_ref, ue_ref, de_ref,
          o_ref):
    f = pl.program_id(1)
    x = x_ref[...]

    @pl.when(f == 0)
    def _router_and_experts():
      bn_scale = 1.0 / jnp.sqrt(1.0 + BN_EPS)
      tm_full = x.shape[0]
      rc = min(256, tm_full)
      for kc in range(tm_full // rc):
        xs = x[kc * rc:(kc + 1) * rc, :]
        lx = jax.lax.dot_general(
            xs, rx_ref[...], (((1,), (1,)), ((), ())),
            preferred_element_type=jnp.float32) * bn_scale
        ly = jax.lax.dot_general(
            xs, ry_ref[...], (((1,), (1,)), ((), ())),
            preferred_element_type=jnp.float32) * bn_scale
        mx = jnp.max(lx, axis=-1, keepdims=True)
        lpx = (lx - mx) - jnp.log(
            jnp.sum(jnp.exp(lx - mx), axis=-1, keepdims=True))
        my = jnp.max(ly, axis=-1, keepdims=True)
        lpy = (ly - my) - jnp.log(
            jnp.sum(jnp.exp(ly - my), axis=-1, keepdims=True))
        c = jnp.concatenate(
            [lpx[:, i:i + 1] + lpy for i in range(NSQ)], axis=-1)
        eidx = jax.lax.broadcasted_iota(jnp.int32, (rc, NSQ * NSQ), 1)
        sel = jnp.zeros((rc, NSQ * NSQ), dtype=jnp.bool_)
        work = c
        for _ in range(TOPK):
            m = jnp.max(work, axis=-1, keepdims=True)
            eq = work == m
            minidx = jnp.min(jnp.where(eq, eidx, NSQ * NSQ),
                             axis=-1, keepdims=True)
            first = eidx == minidx
            sel = sel | first
            work = jnp.where(first, -jnp.inf, work)
        rw = jnp.where(sel, jnp.exp(c), 0.0)
        ax = jax.lax.dot_general(
            xs, ue_ref[...], (((1,), (1,)), ((), ())),
            preferred_element_type=jnp.float32)
        w = jax.nn.silu(ax) * rw
        o_ref[kc * rc:(kc + 1) * rc, :] = jnp.dot(
            w.astype(jnp.bfloat16), de_ref[...],
            preferred_element_type=jnp.float32)

    g = jax.lax.dot_general(x, gw_ref[...], (((1,), (1,)), ((), ())),
                            preferred_element_type=jnp.float32)
    u = jax.lax.dot_general(x, uw_ref[...], (((1,), (1,)), ((), ())),
                            preferred_element_type=jnp.float32)
    h = jax.nn.silu(g) * u
    o_ref[...] += jax.lax.dot_general(h.astype(jnp.bfloat16), dw_ref[...],
                                      (((1,), (1,)), ((), ())),
                                      preferred_element_type=jnp.float32)


def kernel(hidden_states, gate_proj_w, up_proj_w, down_proj_w,
           router_gate_x_w, router_gate_y_w, up_embed, down_embed):
    bsz, seq, d = hidden_states.shape
    t = bsz * seq
    ff = gate_proj_w.shape[0]
    x = hidden_states.reshape(t, d).astype(jnp.bfloat16)
    gw = gate_proj_w.astype(jnp.bfloat16)
    uw = up_proj_w.astype(jnp.bfloat16)
    dw = down_proj_w.astype(jnp.bfloat16)
    rx = router_gate_x_w.astype(jnp.bfloat16)
    ry = router_gate_y_w.astype(jnp.bfloat16)
    ue = up_embed.astype(jnp.bfloat16)
    de = down_embed.astype(jnp.bfloat16)

    tm = min(1024, t)
    fk = min(512, ff)
    n_t = t // tm
    n_ff = ff // fk

    out = pl.pallas_call(
        _body,
        grid=(n_t, n_ff),
        in_specs=[
            pl.BlockSpec((tm, d), lambda i, j: (i, 0)),
            pl.BlockSpec((fk, d), lambda i, j: (j, 0)),
            pl.BlockSpec((fk, d), lambda i, j: (j, 0)),
            pl.BlockSpec((d, fk), lambda i, j: (0, j)),
            pl.BlockSpec((NSQ, d), lambda i, j: (0, 0)),
            pl.BlockSpec((NSQ, d), lambda i, j: (0, 0)),
            pl.BlockSpec((NSQ * NSQ, d), lambda i, j: (0, 0)),
            pl.BlockSpec((NSQ * NSQ, d), lambda i, j: (0, 0)),
        ],
        out_specs=pl.BlockSpec((tm, d), lambda i, j: (i, 0)),
        out_shape=jax.ShapeDtypeStruct((t, d), jnp.float32),
        compiler_params=pltpu.CompilerParams(
            dimension_semantics=("parallel", "arbitrary"),
            vmem_limit_bytes=64 * 1024 * 1024,
        ),
    )(x, gw, uw, dw, rx, ry, ue, de)
    return out.reshape(bsz, seq, d)


# speedup vs baseline: 1.9681x; 1.1488x over previous
import jax
import jax.numpy as jnp
from jax.experimental import pallas as pl
from jax.experimental.pallas import tpu as pltpu

NSQ = 8
TOPK = 8
BN_EPS = 1e-5


def _body(x_ref, gw_ref, uw_ref, dw_ref, rb_ref, de_ref, o_ref):
    f = pl.program_id(1)
    x = x_ref[...]

    @pl.when(f == 0)
    def _router_and_experts():
      bn_scale = 1.0 / jnp.sqrt(1.0 + BN_EPS)
      tm_full = x.shape[0]
      rc = min(256, tm_full)
      for kc in range(tm_full // rc):
        xs = x[kc * rc:(kc + 1) * rc, :]
        logits = jax.lax.dot_general(
            xs, rb_ref[...], (((1,), (1,)), ((), ())),
            preferred_element_type=jnp.float32)
        lx = logits[:, :NSQ] * bn_scale
        ly = logits[:, NSQ:2 * NSQ] * bn_scale
        ax = logits[:, 2 * NSQ:]
        mx = jnp.max(lx, axis=-1, keepdims=True)
        lpx = (lx - mx) - jnp.log(
            jnp.sum(jnp.exp(lx - mx), axis=-1, keepdims=True))
        my = jnp.max(ly, axis=-1, keepdims=True)
        lpy = (ly - my) - jnp.log(
            jnp.sum(jnp.exp(ly - my), axis=-1, keepdims=True))
        c = jnp.concatenate(
            [lpx[:, i:i + 1] + lpy for i in range(NSQ)], axis=-1)
        eidx = jax.lax.broadcasted_iota(jnp.int32, (rc, NSQ * NSQ), 1)
        sel = jnp.zeros((rc, NSQ * NSQ), dtype=jnp.bool_)
        work = c
        for _ in range(TOPK):
            m = jnp.max(work, axis=-1, keepdims=True)
            eq = work == m
            minidx = jnp.min(jnp.where(eq, eidx, NSQ * NSQ),
                             axis=-1, keepdims=True)
            first = eidx == minidx
            sel = sel | first
            work = jnp.where(first, -jnp.inf, work)
        rw = jnp.where(sel, jnp.exp(c), 0.0)
        w = jax.nn.silu(ax) * rw
        o_ref[kc * rc:(kc + 1) * rc, :] = jnp.dot(
            w.astype(jnp.bfloat16), de_ref[...],
            preferred_element_type=jnp.float32)

    g = jax.lax.dot_general(x, gw_ref[...], (((1,), (1,)), ((), ())),
                            preferred_element_type=jnp.float32)
    u = jax.lax.dot_general(x, uw_ref[...], (((1,), (1,)), ((), ())),
                            preferred_element_type=jnp.float32)
    h = jax.nn.silu(g) * u
    o_ref[...] += jax.lax.dot_general(h.astype(jnp.bfloat16),
                                      dw_ref[...].astype(jnp.bfloat16),
                                      (((1,), (1,)), ((), ())),
                                      preferred_element_type=jnp.float32)


def kernel(hidden_states, gate_proj_w, up_proj_w, down_proj_w,
           router_gate_x_w, router_gate_y_w, up_embed, down_embed):
    bsz, seq, d = hidden_states.shape
    t = bsz * seq
    ff = gate_proj_w.shape[0]
    x = hidden_states.reshape(t, d).astype(jnp.bfloat16)
    rb = jnp.concatenate(
        [router_gate_x_w, router_gate_y_w, up_embed], axis=0
    ).astype(jnp.bfloat16)
    de = down_embed.astype(jnp.bfloat16)

    tm = min(1024, t)
    fk = min(512, ff)
    n_t = t // tm
    n_ff = ff // fk

    out = pl.pallas_call(
        _body,
        grid=(n_t, n_ff),
        in_specs=[
            pl.BlockSpec((tm, d), lambda i, j: (i, 0)),
            pl.BlockSpec((fk, d), lambda i, j: (j, 0)),
            pl.BlockSpec((fk, d), lambda i, j: (j, 0)),
            pl.BlockSpec((d, fk), lambda i, j: (0, j)),
            pl.BlockSpec((2 * NSQ + NSQ * NSQ, d),
                         lambda i, j: (0, 0)),
            pl.BlockSpec((NSQ * NSQ, d), lambda i, j: (0, 0)),
        ],
        out_specs=pl.BlockSpec((tm, d), lambda i, j: (i, 0)),
        out_shape=jax.ShapeDtypeStruct((t, d), jnp.float32),
        compiler_params=pltpu.CompilerParams(
            dimension_semantics=("parallel", "arbitrary"),
            vmem_limit_bytes=64 * 1024 * 1024,
        ),
    )(x, gate_proj_w, up_proj_w, down_proj_w, rb, de)
    return out.reshape(bsz, seq, d)


# speedup vs baseline: 1.9878x; 1.0100x over previous
import jax
import jax.numpy as jnp
from jax.experimental import pallas as pl
from jax.experimental.pallas import tpu as pltpu

NSQ = 8
TOPK = 8
BN_EPS = 1e-5


def _body(x_ref, gw_ref, uw_ref, dw_ref, rb_ref, de_ref, o_ref):
    f = pl.program_id(1)
    x = x_ref[...]

    @pl.when(f == 0)
    def _router_and_experts():
      bn_scale = 1.0 / jnp.sqrt(1.0 + BN_EPS)
      tm_full = x.shape[0]
      rc = min(256, tm_full)
      for kc in range(tm_full // rc):
        xs = x[kc * rc:(kc + 1) * rc, :]
        logits = jax.lax.dot_general(
            xs, rb_ref[...], (((1,), (1,)), ((), ())),
            preferred_element_type=jnp.float32)
        lx = logits[:, :NSQ] * bn_scale
        ly = logits[:, NSQ:2 * NSQ] * bn_scale
        ax = logits[:, 2 * NSQ:]
        mx = jnp.max(lx, axis=-1, keepdims=True)
        lpx = (lx - mx) - jnp.log(
            jnp.sum(jnp.exp(lx - mx), axis=-1, keepdims=True))
        my = jnp.max(ly, axis=-1, keepdims=True)
        lpy = (ly - my) - jnp.log(
            jnp.sum(jnp.exp(ly - my), axis=-1, keepdims=True))
        c = jnp.concatenate(
            [lpx[:, i:i + 1] + lpy for i in range(NSQ)], axis=-1)
        eidx = jax.lax.broadcasted_iota(jnp.int32, (rc, NSQ * NSQ), 1)
        sel = jnp.zeros((rc, NSQ * NSQ), dtype=jnp.bool_)
        work = c
        for _ in range(TOPK):
            m = jnp.max(work, axis=-1, keepdims=True)
            eq = work == m
            minidx = jnp.min(jnp.where(eq, eidx, NSQ * NSQ),
                             axis=-1, keepdims=True)
            first = eidx == minidx
            sel = sel | first
            work = jnp.where(first, -jnp.inf, work)
        rw = jnp.where(sel, jnp.exp(c), 0.0)
        w = jax.nn.silu(ax) * rw
        o_ref[kc * rc:(kc + 1) * rc, :] = jnp.dot(
            w.astype(jnp.bfloat16), de_ref[...],
            preferred_element_type=jnp.float32)

    g = jax.lax.dot_general(x, gw_ref[...], (((1,), (1,)), ((), ())),
                            preferred_element_type=jnp.float32)
    u = jax.lax.dot_general(x, uw_ref[...], (((1,), (1,)), ((), ())),
                            preferred_element_type=jnp.float32)
    h = jax.nn.silu(g) * u
    o_ref[...] += jax.lax.dot_general(h, dw_ref[...],
                                      (((1,), (1,)), ((), ())),
                                      preferred_element_type=jnp.float32)


def kernel(hidden_states, gate_proj_w, up_proj_w, down_proj_w,
           router_gate_x_w, router_gate_y_w, up_embed, down_embed):
    bsz, seq, d = hidden_states.shape
    t = bsz * seq
    ff = gate_proj_w.shape[0]
    x = hidden_states.reshape(t, d).astype(jnp.bfloat16)
    rb = jnp.concatenate(
        [router_gate_x_w, router_gate_y_w, up_embed], axis=0
    ).astype(jnp.bfloat16)
    de = down_embed.astype(jnp.bfloat16)

    tm = min(1024, t)
    fk = min(512, ff)
    n_t = t // tm
    n_ff = ff // fk

    out = pl.pallas_call(
        _body,
        grid=(n_t, n_ff),
        in_specs=[
            pl.BlockSpec((tm, d), lambda i, j: (i, 0)),
            pl.BlockSpec((fk, d), lambda i, j: (j, 0)),
            pl.BlockSpec((fk, d), lambda i, j: (j, 0)),
            pl.BlockSpec((d, fk), lambda i, j: (0, j)),
            pl.BlockSpec((2 * NSQ + NSQ * NSQ, d),
                         lambda i, j: (0, 0)),
            pl.BlockSpec((NSQ * NSQ, d), lambda i, j: (0, 0)),
        ],
        out_specs=pl.BlockSpec((tm, d), lambda i, j: (i, 0)),
        out_shape=jax.ShapeDtypeStruct((t, d), jnp.float32),
        compiler_params=pltpu.CompilerParams(
            dimension_semantics=("parallel", "arbitrary"),
            vmem_limit_bytes=64 * 1024 * 1024,
        ),
    )(x, gate_proj_w, up_proj_w, down_proj_w, rb, de)
    return out.reshape(bsz, seq, d)
